# baseline (speedup 1.0000x reference)
import functools

import jax
import jax.numpy as jnp
from jax import lax
from jax.experimental import pallas as pl
from jax.experimental.pallas import tpu as pltpu
from jax.experimental.pallas import tpu_sc as plsc

N = 10000
E = 320000
F_IN = 128
H = 128
C = 10
G = 64

NC = 2
NS = 16
NW = NC * NS
EW = E // NW
K = 80
CH = EW // K
KA = 128
EWA = 10240
NBA = 4
CBA = 20
EPAD = NW * EWA - E
NP = 10112
ROWS_W = NP // NS

_HIGH = lax.Precision.HIGHEST


def _dot(a, b):
    return lax.dot_general(a, b, (((1,), (0,)), ((), ())),
                           precision=_HIGH, preferred_element_type=jnp.float32)


def _sc_mesh():
    return plsc.VectorSubcoreMesh(core_axis_name="c", subcore_axis_name="s")


@jax.jit
def _sc_degree(dst3, ones16, zeros16):
    @functools.partial(
        pl.kernel,
        out_type=jax.ShapeDtypeStruct((NC, NP, H), jnp.float32),
        mesh=_sc_mesh(),
        scratch_types=[
            pltpu.VMEM_SHARED((NP, H), jnp.float32),
            pltpu.VMEM((CH, K), jnp.int32),
            pltpu.VMEM((K, H), jnp.float32),
        ],
    )
    def deg_kernel(dst_hbm, ones_hbm, zeros_hbm, out_hbm, acc, didx, ones_v):
        cid = lax.axis_index("c")
        sid = lax.axis_index("s")
        wid = sid * NC + cid

        pltpu.sync_copy(zeros_hbm.at[pl.ds(sid * ROWS_W, ROWS_W)],
                        acc.at[pl.ds(sid * ROWS_W, ROWS_W)])
        pltpu.sync_copy(dst_hbm.at[wid], didx)
        pltpu.sync_copy(ones_hbm, ones_v)
        plsc.subcore_barrier()

        @pl.loop(0, CH)
        def _(c):
            pltpu.sync_copy(ones_v, acc.at[didx.at[c]], add=True)

        plsc.subcore_barrier()
        pltpu.sync_copy(acc.at[pl.ds(sid * ROWS_W, ROWS_W)],
                        out_hbm.at[cid, pl.ds(sid * ROWS_W, ROWS_W)])

    return deg_kernel(dst3, ones16, zeros16)


@jax.jit
def _sc_aggregate(hp, edge_blk, zeros):
    @functools.partial(
        pl.kernel,
        out_type=jax.ShapeDtypeStruct((NC, NP, H), jnp.float32),
        mesh=_sc_mesh(),
        scratch_types=[
            pltpu.VMEM_SHARED((NP, H), jnp.float32),
            pltpu.VMEM((CBA, 2, KA), jnp.int32),
            pltpu.VMEM((CBA, 2, KA), jnp.int32),
            pltpu.VMEM((KA, H), jnp.float32),
            pltpu.VMEM((KA, H), jnp.float32),
            pltpu.SemaphoreType.DMA,
            pltpu.SemaphoreType.DMA,
            pltpu.SemaphoreType.DMA,
        ],
    )
    def agg_kernel(hp_hbm, edge_hbm, zeros_hbm, out_hbm,
                   acc, iba, ibb, bufa, bufb, sema, semb, semi):
        cid = lax.axis_index("c")
        sid = lax.axis_index("s")
        wid = sid * NC + cid

        pltpu.sync_copy(zeros_hbm.at[pl.ds(sid * ROWS_W, ROWS_W)],
                        acc.at[pl.ds(sid * ROWS_W, ROWS_W)])
        pltpu.sync_copy(edge_hbm.at[wid, 0], iba)
        plsc.subcore_barrier()

        def fire(ib, j, buf, sem):
            return pltpu.async_copy(hp_hbm.at[ib.at[j, 0]], buf, sem)

        def wait_g(ib, j, buf, sem):
            pltpu.make_async_copy(hp_hbm.at[ib.at[j, 0]], buf, sem).wait()

        def scat(ib, j, buf):
            pltpu.sync_copy(buf, acc.at[ib.at[j, 1]], add=True)

        for b in range(NBA):
            ib, ibn = (iba, ibb) if b % 2 == 0 else (ibb, iba)
            if b + 1 < NBA:
                pltpu.async_copy(edge_hbm.at[wid, b + 1], ibn, semi)
            fire(ib, 0, bufa, sema)
            fire(ib, 1, bufb, semb)

            @pl.loop(0, CBA, step=2)
            def _(j, ib=ib):
                wait_g(ib, j, bufa, sema)
                scat(ib, j, bufa)

                @pl.when(j + 2 < CBA)
                def _():
                    fire(ib, j + 2, bufa, sema)

                wait_g(ib, j + 1, bufb, semb)
                scat(ib, j + 1, bufb)

                @pl.when(j + 3 < CBA)
                def _():
                    fire(ib, j + 3, bufb, semb)

            if b + 1 < NBA:
                pltpu.make_async_copy(edge_hbm.at[wid, b + 1], ibn, semi).wait()

        plsc.subcore_barrier()
        pltpu.sync_copy(acc.at[pl.ds(sid * ROWS_W, ROWS_W)],
                        out_hbm.at[cid, pl.ds(sid * ROWS_W, ROWS_W)])

    return agg_kernel(hp, edge_blk, zeros)


_R = 1000


def _tc_matmul(x, W):
    def body(x_ref, w_ref, o_ref):
        o_ref[...] = _dot(x_ref[...], w_ref[...])

    return pl.pallas_call(
        body,
        grid=(N // _R,),
        in_specs=[pl.BlockSpec((_R, F_IN), lambda i: (i, 0)),
                  pl.BlockSpec((F_IN, H), lambda i: (0, 0))],
        out_specs=pl.BlockSpec((_R, H), lambda i: (i, 0)),
        out_shape=jax.ShapeDtypeStruct((N, H), jnp.float32),
    )(x, W)


def _tc_prescale(t1, degp):
    def body(t_ref, d_ref, hp_ref, dis_ref):
        deg = d_ref[0, :, 0:1] + d_ref[1, :, 0:1] + 1.0
        dis = lax.rsqrt(deg)
        dis_ref[...] = dis
        hp_ref[...] = t_ref[...] * dis

    return pl.pallas_call(
        body,
        grid=(N // _R,),
        in_specs=[pl.BlockSpec((_R, H), lambda i: (i, 0)),
                  pl.BlockSpec((NC, _R, H), lambda i: (0, i, 0))],
        out_specs=[pl.BlockSpec((_R, H), lambda i: (i, 0)),
                   pl.BlockSpec((_R, 1), lambda i: (i, 0))],
        out_shape=[jax.ShapeDtypeStruct((N, H), jnp.float32),
                   jax.ShapeDtypeStruct((N, 1), jnp.float32)],
    )(t1, degp)


def _tc_layer(parts, hp, dis, b, W, relu=True):
    def body(p_ref, hp_ref, dis_ref, b_ref, w_ref, o_ref):
        dis = dis_ref[...]
        z = dis * (p_ref[0] + p_ref[1] + hp_ref[...]) + b_ref[...]
        if relu:
            z = jnp.maximum(z, 0.0)
        o_ref[...] = _dot(z, w_ref[...]) * dis

    return pl.pallas_call(
        body,
        grid=(N // _R,),
        in_specs=[pl.BlockSpec((NC, _R, H), lambda i: (0, i, 0)),
                  pl.BlockSpec((_R, H), lambda i: (i, 0)),
                  pl.BlockSpec((_R, 1), lambda i: (i, 0)),
                  pl.BlockSpec((1, H), lambda i: (0, 0)),
                  pl.BlockSpec((H, H), lambda i: (0, 0))],
        out_specs=pl.BlockSpec((_R, H), lambda i: (i, 0)),
        out_shape=jax.ShapeDtypeStruct((N, H), jnp.float32),
    )(parts, hp, dis, b, W)


def _tc_pool_head(parts, hp, dis, b3, batch2, Wl, bl):
    def body(p_ref, hp_ref, dis_ref, b_ref, bat_ref, wl_ref, bl_ref, o_ref):
        z = dis_ref[...] * (p_ref[0] + p_ref[1] + hp_ref[...]) + b_ref[...]
        gids = lax.broadcasted_iota(jnp.int32, (G, N), 0)
        mask = (bat_ref[...] == gids).astype(jnp.float32)
        sums = _dot(mask, z)
        counts = jnp.sum(mask, axis=1, keepdims=True)
        pooled = sums / jnp.maximum(counts, 1.0)
        o_ref[...] = _dot(pooled, wl_ref[...]) + bl_ref[...]

    return pl.pallas_call(
        body,
        grid=(1,),
        in_specs=[pl.BlockSpec((NC, N, H), lambda i: (0, 0, 0)),
                  pl.BlockSpec((N, H), lambda i: (0, 0)),
                  pl.BlockSpec((N, 1), lambda i: (0, 0)),
                  pl.BlockSpec((1, H), lambda i: (0, 0)),
                  pl.BlockSpec((1, N), lambda i: (0, 0)),
                  pl.BlockSpec((H, C), lambda i: (0, 0)),
                  pl.BlockSpec((1, C), lambda i: (0, 0))],
        out_specs=pl.BlockSpec((G, C), lambda i: (0, 0)),
        out_shape=jax.ShapeDtypeStruct((G, C), jnp.float32),
    )(parts, hp, dis, b3, batch2, Wl, bl)


def kernel(x, edge_index, batch, W1, b1, W2, b2, W3, b3, Wl, bl):
    dst3 = edge_index[1].reshape(NW, CH, K)
    src_p = jnp.concatenate([edge_index[0], jnp.zeros((EPAD,), jnp.int32)])
    junk = N + (jnp.arange(EPAD, dtype=jnp.int32) % (NP - N))
    dst_p = jnp.concatenate([edge_index[1], junk])
    edge_blk = jnp.stack([src_p, dst_p]).reshape(
        2, NW, NBA, CBA, KA).transpose(1, 2, 3, 0, 4)
    zeros = jnp.zeros((NP, H), jnp.float32)
    ones16 = jnp.ones((K, H), jnp.float32)
    batch2 = batch.reshape(1, N)
    b1r, b2r, b3r = b1.reshape(1, H), b2.reshape(1, H), b3.reshape(1, H)
    blr = bl.reshape(1, C)

    degp = _sc_degree(dst3, ones16, zeros)
    t1 = _tc_matmul(x, W1)
    h1p, dis = _tc_prescale(t1, degp)

    p1 = _sc_aggregate(h1p, edge_blk, zeros)
    h2p = _tc_layer(p1, h1p, dis, b1r, W2, relu=True)

    p2 = _sc_aggregate(h2p, edge_blk, zeros)
    h3p = _tc_layer(p2, h2p, dis, b2r, W3, relu=True)

    p3 = _sc_aggregate(h3p, edge_blk, zeros)
    return _tc_pool_head(p3, h3p, dis, b3r, batch2, Wl, blr)


# speedup vs baseline: 2.7982x; 2.7982x over previous
import functools

import jax
import jax.numpy as jnp
from jax import lax
from jax.experimental import pallas as pl
from jax.experimental.pallas import tpu as pltpu
from jax.experimental.pallas import tpu_sc as plsc

N = 10000
E = 320000
F_IN = 128
H = 128
C = 10
G = 64

NC = 2
NS = 16
NW = NC * NS
EW = E // NW
K = 80
CH = EW // K
NB = 5
CB = CH // NB
NP = 10112
ROWS_W = NP // NS

_HIGH = lax.Precision.HIGHEST


def _dot(a, b):
    return lax.dot_general(a, b, (((1,), (0,)), ((), ())),
                           precision=_HIGH, preferred_element_type=jnp.float32)


def _sc_mesh():
    return plsc.VectorSubcoreMesh(core_axis_name="c", subcore_axis_name="s")


@jax.jit
def _sc_degree(dst3, ones16, zeros16):
    @functools.partial(
        pl.kernel,
        out_type=jax.ShapeDtypeStruct((NC, NP, H), jnp.float32),
        mesh=_sc_mesh(),
        scratch_types=[
            pltpu.VMEM_SHARED((NP, H), jnp.float32),
            pltpu.VMEM((CH, K), jnp.int32),
            pltpu.VMEM((K, H), jnp.float32),
        ],
    )
    def deg_kernel(dst_hbm, ones_hbm, zeros_hbm, out_hbm, acc, didx, ones_v):
        cid = lax.axis_index("c")
        sid = lax.axis_index("s")
        wid = sid * NC + cid

        pltpu.sync_copy(zeros_hbm.at[pl.ds(sid * ROWS_W, ROWS_W)],
                        acc.at[pl.ds(sid * ROWS_W, ROWS_W)])
        pltpu.sync_copy(dst_hbm.at[wid], didx)
        pltpu.sync_copy(ones_hbm, ones_v)
        plsc.subcore_barrier()

        @pl.loop(0, CH)
        def _(c):
            pltpu.sync_copy(ones_v, acc.at[didx.at[c]], add=True)

        plsc.subcore_barrier()
        pltpu.sync_copy(acc.at[pl.ds(sid * ROWS_W, ROWS_W)],
                        out_hbm.at[cid, pl.ds(sid * ROWS_W, ROWS_W)])

    return deg_kernel(dst3, ones16, zeros16)


@jax.jit
def _sc_aggregate(hp, edge_blk, zeros):
    @functools.partial(
        pl.kernel,
        out_type=jax.ShapeDtypeStruct((NC, NP, H), jnp.float32),
        mesh=_sc_mesh(),
        scratch_types=[
            pltpu.VMEM_SHARED((NP, H), jnp.float32),
            pltpu.VMEM((CB, 2, K), jnp.int32),
            pltpu.VMEM((CB, 2, K), jnp.int32),
            pltpu.VMEM((K, H), jnp.float32),
            pltpu.VMEM((K, H), jnp.float32),
            pltpu.SemaphoreType.DMA,
            pltpu.SemaphoreType.DMA,
            pltpu.SemaphoreType.DMA,
        ],
    )
    def agg_kernel(hp_hbm, edge_hbm, zeros_hbm, out_hbm,
                   acc, iba, ibb, bufa, bufb, sema, semb, semi):
        cid = lax.axis_index("c")
        sid = lax.axis_index("s")
        wid = sid * NC + cid

        pltpu.sync_copy(zeros_hbm.at[pl.ds(sid * ROWS_W, ROWS_W)],
                        acc.at[pl.ds(sid * ROWS_W, ROWS_W)])
        pltpu.sync_copy(edge_hbm.at[wid, 0], iba)
        plsc.subcore_barrier()

        def fire(ib, j, buf, sem):
            return pltpu.async_copy(hp_hbm.at[ib.at[j, 0]], buf, sem)

        def wait_g(ib, j, buf, sem):
            pltpu.make_async_copy(hp_hbm.at[ib.at[j, 0]], buf, sem).wait()

        def scat(ib, j, buf):
            pltpu.sync_copy(buf, acc.at[ib.at[j, 1]], add=True)

        for b in range(NB):
            ib, ibn = (iba, ibb) if b % 2 == 0 else (ibb, iba)
            if b + 1 < NB:
                pltpu.async_copy(edge_hbm.at[wid, b + 1], ibn, semi)
            fire(ib, 0, bufa, sema)
            fire(ib, 1, bufb, semb)

            @pl.loop(0, CB, step=2)
            def _(j, ib=ib):
                wait_g(ib, j, bufa, sema)
                scat(ib, j, bufa)

                @pl.when(j + 2 < CB)
                def _():
                    fire(ib, j + 2, bufa, sema)

                @pl.when(j + 1 < CB)
                def _():
                    wait_g(ib, j + 1, bufb, semb)
                    scat(ib, j + 1, bufb)

                    @pl.when(j + 3 < CB)
                    def _():
                        fire(ib, j + 3, bufb, semb)

            if b + 1 < NB:
                pltpu.make_async_copy(edge_hbm.at[wid, b + 1], ibn, semi).wait()

        plsc.subcore_barrier()
        pltpu.sync_copy(acc.at[pl.ds(sid * ROWS_W, ROWS_W)],
                        out_hbm.at[cid, pl.ds(sid * ROWS_W, ROWS_W)])

    return agg_kernel(hp, edge_blk, zeros)


_R = 1000


def _tc_matmul(x, W):
    def body(x_ref, w_ref, o_ref):
        o_ref[...] = _dot(x_ref[...], w_ref[...])

    return pl.pallas_call(
        body,
        grid=(N // _R,),
        in_specs=[pl.BlockSpec((_R, F_IN), lambda i: (i, 0)),
                  pl.BlockSpec((F_IN, H), lambda i: (0, 0))],
        out_specs=pl.BlockSpec((_R, H), lambda i: (i, 0)),
        out_shape=jax.ShapeDtypeStruct((N, H), jnp.float32),
    )(x, W)


def _tc_prescale(t1, degp):
    def body(t_ref, d_ref, hp_ref, dis_ref):
        deg = d_ref[0, :, 0:1] + d_ref[1, :, 0:1] + 1.0
        dis = lax.rsqrt(deg)
        dis_ref[...] = dis
        hp_ref[...] = t_ref[...] * dis

    return pl.pallas_call(
        body,
        grid=(N // _R,),
        in_specs=[pl.BlockSpec((_R, H), lambda i: (i, 0)),
                  pl.BlockSpec((NC, _R, H), lambda i: (0, i, 0))],
        out_specs=[pl.BlockSpec((_R, H), lambda i: (i, 0)),
                   pl.BlockSpec((_R, 1), lambda i: (i, 0))],
        out_shape=[jax.ShapeDtypeStruct((N, H), jnp.float32),
                   jax.ShapeDtypeStruct((N, 1), jnp.float32)],
    )(t1, degp)


def _tc_layer(parts, hp, dis, b, W, relu=True):
    def body(p_ref, hp_ref, dis_ref, b_ref, w_ref, o_ref):
        dis = dis_ref[...]
        z = dis * (p_ref[0] + p_ref[1] + hp_ref[...]) + b_ref[...]
        if relu:
            z = jnp.maximum(z, 0.0)
        o_ref[...] = _dot(z, w_ref[...]) * dis

    return pl.pallas_call(
        body,
        grid=(N // _R,),
        in_specs=[pl.BlockSpec((NC, _R, H), lambda i: (0, i, 0)),
                  pl.BlockSpec((_R, H), lambda i: (i, 0)),
                  pl.BlockSpec((_R, 1), lambda i: (i, 0)),
                  pl.BlockSpec((1, H), lambda i: (0, 0)),
                  pl.BlockSpec((H, H), lambda i: (0, 0))],
        out_specs=pl.BlockSpec((_R, H), lambda i: (i, 0)),
        out_shape=jax.ShapeDtypeStruct((N, H), jnp.float32),
    )(parts, hp, dis, b, W)


def _tc_pool_head(parts, hp, dis, b3, batch2, Wl, bl):
    def body(p_ref, hp_ref, dis_ref, b_ref, bat_ref, wl_ref, bl_ref, o_ref):
        z = dis_ref[...] * (p_ref[0] + p_ref[1] + hp_ref[...]) + b_ref[...]
        gids = lax.broadcasted_iota(jnp.int32, (G, N), 0)
        mask = (bat_ref[...] == gids).astype(jnp.float32)
        sums = _dot(mask, z)
        counts = jnp.sum(mask, axis=1, keepdims=True)
        pooled = sums / jnp.maximum(counts, 1.0)
        o_ref[...] = _dot(pooled, wl_ref[...]) + bl_ref[...]

    return pl.pallas_call(
        body,
        grid=(1,),
        in_specs=[pl.BlockSpec((NC, N, H), lambda i: (0, 0, 0)),
                  pl.BlockSpec((N, H), lambda i: (0, 0)),
                  pl.BlockSpec((N, 1), lambda i: (0, 0)),
                  pl.BlockSpec((1, H), lambda i: (0, 0)),
                  pl.BlockSpec((1, N), lambda i: (0, 0)),
                  pl.BlockSpec((H, C), lambda i: (0, 0)),
                  pl.BlockSpec((1, C), lambda i: (0, 0))],
        out_specs=pl.BlockSpec((G, C), lambda i: (0, 0)),
        out_shape=jax.ShapeDtypeStruct((G, C), jnp.float32),
    )(parts, hp, dis, b3, batch2, Wl, bl)


def kernel(x, edge_index, batch, W1, b1, W2, b2, W3, b3, Wl, bl):
    dst3 = edge_index[1].reshape(NW, CH, K)
    edge_blk = edge_index.reshape(2, NW, NB, CB, K).transpose(1, 2, 3, 0, 4)
    zeros = jnp.zeros((NP, H), jnp.float32)
    ones16 = jnp.ones((K, H), jnp.float32)
    batch2 = batch.reshape(1, N)
    b1r, b2r, b3r = b1.reshape(1, H), b2.reshape(1, H), b3.reshape(1, H)
    blr = bl.reshape(1, C)

    degp = _sc_degree(dst3, ones16, zeros)
    t1 = _tc_matmul(x, W1)
    h1p, dis = _tc_prescale(t1, degp)

    p1 = _sc_aggregate(h1p, edge_blk, zeros)
    h2p = _tc_layer(p1, h1p, dis, b1r, W2, relu=True)

    p2 = _sc_aggregate(h2p, edge_blk, zeros)
    h3p = _tc_layer(p2, h2p, dis, b2r, W3, relu=True)

    p3 = _sc_aggregate(h3p, edge_blk, zeros)
    return _tc_pool_head(p3, h3p, dis, b3r, batch2, Wl, blr)


# speedup vs baseline: 2.8882x; 1.0322x over previous
import functools

import jax
import jax.numpy as jnp
from jax import lax
from jax.experimental import pallas as pl
from jax.experimental.pallas import tpu as pltpu
from jax.experimental.pallas import tpu_sc as plsc

N = 10000
E = 320000
F_IN = 128
H = 128
C = 10
G = 64

NC = 2
NS = 16
NW = NC * NS
EW = E // NW
K = 100
CH = EW // K
NB = 5
CB = CH // NB
NP = 10112
ROWS_W = NP // NS

_HIGH = lax.Precision.HIGHEST


def _dot(a, b):
    return lax.dot_general(a, b, (((1,), (0,)), ((), ())),
                           precision=_HIGH, preferred_element_type=jnp.float32)


def _sc_mesh():
    return plsc.VectorSubcoreMesh(core_axis_name="c", subcore_axis_name="s")


@jax.jit
def _sc_degree(dst3, ones16, zeros16):
    @functools.partial(
        pl.kernel,
        out_type=jax.ShapeDtypeStruct((NC, NP, H), jnp.float32),
        mesh=_sc_mesh(),
        scratch_types=[
            pltpu.VMEM_SHARED((NP, H), jnp.float32),
            pltpu.VMEM((CH, K), jnp.int32),
            pltpu.VMEM((K, H), jnp.float32),
        ],
    )
    def deg_kernel(dst_hbm, ones_hbm, zeros_hbm, out_hbm, acc, didx, ones_v):
        cid = lax.axis_index("c")
        sid = lax.axis_index("s")
        wid = sid * NC + cid

        pltpu.sync_copy(zeros_hbm.at[pl.ds(sid * ROWS_W, ROWS_W)],
                        acc.at[pl.ds(sid * ROWS_W, ROWS_W)])
        pltpu.sync_copy(dst_hbm.at[wid], didx)
        pltpu.sync_copy(ones_hbm, ones_v)
        plsc.subcore_barrier()

        @pl.loop(0, CH)
        def _(c):
            pltpu.sync_copy(ones_v, acc.at[didx.at[c]], add=True)

        plsc.subcore_barrier()
        pltpu.sync_copy(acc.at[pl.ds(sid * ROWS_W, ROWS_W)],
                        out_hbm.at[cid, pl.ds(sid * ROWS_W, ROWS_W)])

    return deg_kernel(dst3, ones16, zeros16)


@jax.jit
def _sc_aggregate(hp, edge_blk, zeros):
    @functools.partial(
        pl.kernel,
        out_type=jax.ShapeDtypeStruct((NC, NP, H), jnp.float32),
        mesh=_sc_mesh(),
        scratch_types=[
            pltpu.VMEM_SHARED((NP, H), jnp.float32),
            pltpu.VMEM((CB, 2, K), jnp.int32),
            pltpu.VMEM((CB, 2, K), jnp.int32),
            pltpu.VMEM((K, H), jnp.float32),
            pltpu.VMEM((K, H), jnp.float32),
            pltpu.SemaphoreType.DMA,
            pltpu.SemaphoreType.DMA,
            pltpu.SemaphoreType.DMA,
        ],
    )
    def agg_kernel(hp_hbm, edge_hbm, zeros_hbm, out_hbm,
                   acc, iba, ibb, bufa, bufb, sema, semb, semi):
        cid = lax.axis_index("c")
        sid = lax.axis_index("s")
        wid = sid * NC + cid

        pltpu.sync_copy(zeros_hbm.at[pl.ds(sid * ROWS_W, ROWS_W)],
                        acc.at[pl.ds(sid * ROWS_W, ROWS_W)])
        pltpu.sync_copy(edge_hbm.at[wid, 0], iba)
        plsc.subcore_barrier()

        def fire(ib, j, buf, sem):
            return pltpu.async_copy(hp_hbm.at[ib.at[j, 0]], buf, sem)

        def wait_g(ib, j, buf, sem):
            pltpu.make_async_copy(hp_hbm.at[ib.at[j, 0]], buf, sem).wait()

        def scat(ib, j, buf):
            pltpu.sync_copy(buf, acc.at[ib.at[j, 1]], add=True)

        for b in range(NB):
            ib, ibn = (iba, ibb) if b % 2 == 0 else (ibb, iba)
            if b + 1 < NB:
                pltpu.async_copy(edge_hbm.at[wid, b + 1], ibn, semi)
            fire(ib, 0, bufa, sema)
            fire(ib, 1, bufb, semb)

            @pl.loop(0, CB, step=2)
            def _(j, ib=ib):
                wait_g(ib, j, bufa, sema)
                scat(ib, j, bufa)

                @pl.when(j + 2 < CB)
                def _():
                    fire(ib, j + 2, bufa, sema)

                @pl.when(j + 1 < CB)
                def _():
                    wait_g(ib, j + 1, bufb, semb)
                    scat(ib, j + 1, bufb)

                    @pl.when(j + 3 < CB)
                    def _():
                        fire(ib, j + 3, bufb, semb)

            if b + 1 < NB:
                pltpu.make_async_copy(edge_hbm.at[wid, b + 1], ibn, semi).wait()

        plsc.subcore_barrier()
        pltpu.sync_copy(acc.at[pl.ds(sid * ROWS_W, ROWS_W)],
                        out_hbm.at[cid, pl.ds(sid * ROWS_W, ROWS_W)])

    return agg_kernel(hp, edge_blk, zeros)


_R = 1000


def _tc_matmul(x, W):
    def body(x_ref, w_ref, o_ref):
        o_ref[...] = _dot(x_ref[...], w_ref[...])

    return pl.pallas_call(
        body,
        grid=(N // _R,),
        in_specs=[pl.BlockSpec((_R, F_IN), lambda i: (i, 0)),
                  pl.BlockSpec((F_IN, H), lambda i: (0, 0))],
        out_specs=pl.BlockSpec((_R, H), lambda i: (i, 0)),
        out_shape=jax.ShapeDtypeStruct((N, H), jnp.float32),
    )(x, W)


def _tc_prescale(t1, degp):
    def body(t_ref, d_ref, hp_ref, dis_ref):
        deg = d_ref[0, :, 0:1] + d_ref[1, :, 0:1] + 1.0
        dis = lax.rsqrt(deg)
        dis_ref[...] = dis
        hp_ref[...] = t_ref[...] * dis

    return pl.pallas_call(
        body,
        grid=(N // _R,),
        in_specs=[pl.BlockSpec((_R, H), lambda i: (i, 0)),
                  pl.BlockSpec((NC, _R, H), lambda i: (0, i, 0))],
        out_specs=[pl.BlockSpec((_R, H), lambda i: (i, 0)),
                   pl.BlockSpec((_R, 1), lambda i: (i, 0))],
        out_shape=[jax.ShapeDtypeStruct((N, H), jnp.float32),
                   jax.ShapeDtypeStruct((N, 1), jnp.float32)],
    )(t1, degp)


def _tc_layer(parts, hp, dis, b, W, relu=True):
    def body(p_ref, hp_ref, dis_ref, b_ref, w_ref, o_ref):
        dis = dis_ref[...]
        z = dis * (p_ref[0] + p_ref[1] + hp_ref[...]) + b_ref[...]
        if relu:
            z = jnp.maximum(z, 0.0)
        o_ref[...] = _dot(z, w_ref[...]) * dis

    return pl.pallas_call(
        body,
        grid=(N // _R,),
        in_specs=[pl.BlockSpec((NC, _R, H), lambda i: (0, i, 0)),
                  pl.BlockSpec((_R, H), lambda i: (i, 0)),
                  pl.BlockSpec((_R, 1), lambda i: (i, 0)),
                  pl.BlockSpec((1, H), lambda i: (0, 0)),
                  pl.BlockSpec((H, H), lambda i: (0, 0))],
        out_specs=pl.BlockSpec((_R, H), lambda i: (i, 0)),
        out_shape=jax.ShapeDtypeStruct((N, H), jnp.float32),
    )(parts, hp, dis, b, W)


def _tc_pool_head(parts, hp, dis, b3, batch2, Wl, bl):
    def body(p_ref, hp_ref, dis_ref, b_ref, bat_ref, wl_ref, bl_ref, o_ref):
        z = dis_ref[...] * (p_ref[0] + p_ref[1] + hp_ref[...]) + b_ref[...]
        gids = lax.broadcasted_iota(jnp.int32, (G, N), 0)
        mask = (bat_ref[...] == gids).astype(jnp.float32)
        sums = _dot(mask, z)
        counts = jnp.sum(mask, axis=1, keepdims=True)
        pooled = sums / jnp.maximum(counts, 1.0)
        o_ref[...] = _dot(pooled, wl_ref[...]) + bl_ref[...]

    return pl.pallas_call(
        body,
        grid=(1,),
        in_specs=[pl.BlockSpec((NC, N, H), lambda i: (0, 0, 0)),
                  pl.BlockSpec((N, H), lambda i: (0, 0)),
                  pl.BlockSpec((N, 1), lambda i: (0, 0)),
                  pl.BlockSpec((1, H), lambda i: (0, 0)),
                  pl.BlockSpec((1, N), lambda i: (0, 0)),
                  pl.BlockSpec((H, C), lambda i: (0, 0)),
                  pl.BlockSpec((1, C), lambda i: (0, 0))],
        out_specs=pl.BlockSpec((G, C), lambda i: (0, 0)),
        out_shape=jax.ShapeDtypeStruct((G, C), jnp.float32),
    )(parts, hp, dis, b3, batch2, Wl, bl)


def kernel(x, edge_index, batch, W1, b1, W2, b2, W3, b3, Wl, bl):
    dst3 = edge_index[1].reshape(NW, CH, K)
    edge_blk = edge_index.reshape(2, NW, NB, CB, K).transpose(1, 2, 3, 0, 4)
    zeros = jnp.zeros((NP, H), jnp.float32)
    ones16 = jnp.ones((K, H), jnp.float32)
    batch2 = batch.reshape(1, N)
    b1r, b2r, b3r = b1.reshape(1, H), b2.reshape(1, H), b3.reshape(1, H)
    blr = bl.reshape(1, C)

    degp = _sc_degree(dst3, ones16, zeros)
    t1 = _tc_matmul(x, W1)
    h1p, dis = _tc_prescale(t1, degp)

    p1 = _sc_aggregate(h1p, edge_blk, zeros)
    h2p = _tc_layer(p1, h1p, dis, b1r, W2, relu=True)

    p2 = _sc_aggregate(h2p, edge_blk, zeros)
    h3p = _tc_layer(p2, h2p, dis, b2r, W3, relu=True)

    p3 = _sc_aggregate(h3p, edge_blk, zeros)
    return _tc_pool_head(p3, h3p, dis, b3r, batch2, Wl, blr)


# speedup vs baseline: 2.9509x; 1.0217x over previous
import functools

import jax
import jax.numpy as jnp
from jax import lax
from jax.experimental import pallas as pl
from jax.experimental.pallas import tpu as pltpu
from jax.experimental.pallas import tpu_sc as plsc

N = 10000
E = 320000
F_IN = 128
H = 128
C = 10
G = 64

NC = 2
NS = 16
NW = NC * NS
EW = E // NW
K = 125
CH = EW // K
NB = 5
CB = CH // NB
NP = 10112
ROWS_W = NP // NS

_HIGH = lax.Precision.HIGHEST


def _dot(a, b):
    return lax.dot_general(a, b, (((1,), (0,)), ((), ())),
                           precision=_HIGH, preferred_element_type=jnp.float32)


def _sc_mesh():
    return plsc.VectorSubcoreMesh(core_axis_name="c", subcore_axis_name="s")


@jax.jit
def _sc_degree(dst3, ones16, zeros16):
    @functools.partial(
        pl.kernel,
        out_type=jax.ShapeDtypeStruct((NC, NP, H), jnp.float32),
        mesh=_sc_mesh(),
        scratch_types=[
            pltpu.VMEM_SHARED((NP, H), jnp.float32),
            pltpu.VMEM((CH, K), jnp.int32),
            pltpu.VMEM((K, H), jnp.float32),
        ],
    )
    def deg_kernel(dst_hbm, ones_hbm, zeros_hbm, out_hbm, acc, didx, ones_v):
        cid = lax.axis_index("c")
        sid = lax.axis_index("s")
        wid = sid * NC + cid

        pltpu.sync_copy(zeros_hbm.at[pl.ds(sid * ROWS_W, ROWS_W)],
                        acc.at[pl.ds(sid * ROWS_W, ROWS_W)])
        pltpu.sync_copy(dst_hbm.at[wid], didx)
        pltpu.sync_copy(ones_hbm, ones_v)
        plsc.subcore_barrier()

        @pl.loop(0, CH)
        def _(c):
            pltpu.sync_copy(ones_v, acc.at[didx.at[c]], add=True)

        plsc.subcore_barrier()
        pltpu.sync_copy(acc.at[pl.ds(sid * ROWS_W, ROWS_W)],
                        out_hbm.at[cid, pl.ds(sid * ROWS_W, ROWS_W)])

    return deg_kernel(dst3, ones16, zeros16)


@jax.jit
def _sc_aggregate(hp, edge_blk, zeros):
    @functools.partial(
        pl.kernel,
        out_type=jax.ShapeDtypeStruct((NC, NP, H), jnp.float32),
        mesh=_sc_mesh(),
        scratch_types=[
            pltpu.VMEM_SHARED((NP, H), jnp.float32),
            pltpu.VMEM((CB, 2, K), jnp.int32),
            pltpu.VMEM((CB, 2, K), jnp.int32),
            pltpu.VMEM((K, H), jnp.float32),
            pltpu.VMEM((K, H), jnp.float32),
            pltpu.SemaphoreType.DMA,
            pltpu.SemaphoreType.DMA,
            pltpu.SemaphoreType.DMA,
        ],
    )
    def agg_kernel(hp_hbm, edge_hbm, zeros_hbm, out_hbm,
                   acc, iba, ibb, bufa, bufb, sema, semb, semi):
        cid = lax.axis_index("c")
        sid = lax.axis_index("s")
        wid = sid * NC + cid

        pltpu.sync_copy(zeros_hbm.at[pl.ds(sid * ROWS_W, ROWS_W)],
                        acc.at[pl.ds(sid * ROWS_W, ROWS_W)])
        pltpu.sync_copy(edge_hbm.at[wid, 0], iba)
        plsc.subcore_barrier()

        def fire(ib, j, buf, sem):
            return pltpu.async_copy(hp_hbm.at[ib.at[j, 0]], buf, sem)

        def wait_g(ib, j, buf, sem):
            pltpu.make_async_copy(hp_hbm.at[ib.at[j, 0]], buf, sem).wait()

        def scat(ib, j, buf):
            pltpu.sync_copy(buf, acc.at[ib.at[j, 1]], add=True)

        for b in range(NB):
            ib, ibn = (iba, ibb) if b % 2 == 0 else (ibb, iba)
            if b + 1 < NB:
                pltpu.async_copy(edge_hbm.at[wid, b + 1], ibn, semi)
            fire(ib, 0, bufa, sema)
            fire(ib, 1, bufb, semb)

            @pl.loop(0, CB, step=2)
            def _(j, ib=ib):
                wait_g(ib, j, bufa, sema)
                scat(ib, j, bufa)

                @pl.when(j + 2 < CB)
                def _():
                    fire(ib, j + 2, bufa, sema)

                @pl.when(j + 1 < CB)
                def _():
                    wait_g(ib, j + 1, bufb, semb)
                    scat(ib, j + 1, bufb)

                    @pl.when(j + 3 < CB)
                    def _():
                        fire(ib, j + 3, bufb, semb)

            if b + 1 < NB:
                pltpu.make_async_copy(edge_hbm.at[wid, b + 1], ibn, semi).wait()

        plsc.subcore_barrier()
        pltpu.sync_copy(acc.at[pl.ds(sid * ROWS_W, ROWS_W)],
                        out_hbm.at[cid, pl.ds(sid * ROWS_W, ROWS_W)])

    return agg_kernel(hp, edge_blk, zeros)


_R = 1000


def _tc_matmul(x, W):
    def body(x_ref, w_ref, o_ref):
        o_ref[...] = _dot(x_ref[...], w_ref[...])

    return pl.pallas_call(
        body,
        grid=(N // _R,),
        in_specs=[pl.BlockSpec((_R, F_IN), lambda i: (i, 0)),
                  pl.BlockSpec((F_IN, H), lambda i: (0, 0))],
        out_specs=pl.BlockSpec((_R, H), lambda i: (i, 0)),
        out_shape=jax.ShapeDtypeStruct((N, H), jnp.float32),
    )(x, W)


def _tc_prescale(t1, degp):
    def body(t_ref, d_ref, hp_ref, dis_ref):
        deg = d_ref[0, :, 0:1] + d_ref[1, :, 0:1] + 1.0
        dis = lax.rsqrt(deg)
        dis_ref[...] = dis
        hp_ref[...] = t_ref[...] * dis

    return pl.pallas_call(
        body,
        grid=(N // _R,),
        in_specs=[pl.BlockSpec((_R, H), lambda i: (i, 0)),
                  pl.BlockSpec((NC, _R, H), lambda i: (0, i, 0))],
        out_specs=[pl.BlockSpec((_R, H), lambda i: (i, 0)),
                   pl.BlockSpec((_R, 1), lambda i: (i, 0))],
        out_shape=[jax.ShapeDtypeStruct((N, H), jnp.float32),
                   jax.ShapeDtypeStruct((N, 1), jnp.float32)],
    )(t1, degp)


def _tc_layer(parts, hp, dis, b, W, relu=True):
    def body(p_ref, hp_ref, dis_ref, b_ref, w_ref, o_ref):
        dis = dis_ref[...]
        z = dis * (p_ref[0] + p_ref[1] + hp_ref[...]) + b_ref[...]
        if relu:
            z = jnp.maximum(z, 0.0)
        o_ref[...] = _dot(z, w_ref[...]) * dis

    return pl.pallas_call(
        body,
        grid=(N // _R,),
        in_specs=[pl.BlockSpec((NC, _R, H), lambda i: (0, i, 0)),
                  pl.BlockSpec((_R, H), lambda i: (i, 0)),
                  pl.BlockSpec((_R, 1), lambda i: (i, 0)),
                  pl.BlockSpec((1, H), lambda i: (0, 0)),
                  pl.BlockSpec((H, H), lambda i: (0, 0))],
        out_specs=pl.BlockSpec((_R, H), lambda i: (i, 0)),
        out_shape=jax.ShapeDtypeStruct((N, H), jnp.float32),
    )(parts, hp, dis, b, W)


def _tc_pool_head(parts, hp, dis, b3, batch2, Wl, bl):
    def body(p_ref, hp_ref, dis_ref, b_ref, bat_ref, wl_ref, bl_ref, o_ref):
        z = dis_ref[...] * (p_ref[0] + p_ref[1] + hp_ref[...]) + b_ref[...]
        gids = lax.broadcasted_iota(jnp.int32, (G, N), 0)
        mask = (bat_ref[...] == gids).astype(jnp.float32)
        sums = _dot(mask, z)
        counts = jnp.sum(mask, axis=1, keepdims=True)
        pooled = sums / jnp.maximum(counts, 1.0)
        o_ref[...] = _dot(pooled, wl_ref[...]) + bl_ref[...]

    return pl.pallas_call(
        body,
        grid=(1,),
        in_specs=[pl.BlockSpec((NC, N, H), lambda i: (0, 0, 0)),
                  pl.BlockSpec((N, H), lambda i: (0, 0)),
                  pl.BlockSpec((N, 1), lambda i: (0, 0)),
                  pl.BlockSpec((1, H), lambda i: (0, 0)),
                  pl.BlockSpec((1, N), lambda i: (0, 0)),
                  pl.BlockSpec((H, C), lambda i: (0, 0)),
                  pl.BlockSpec((1, C), lambda i: (0, 0))],
        out_specs=pl.BlockSpec((G, C), lambda i: (0, 0)),
        out_shape=jax.ShapeDtypeStruct((G, C), jnp.float32),
    )(parts, hp, dis, b3, batch2, Wl, bl)


def kernel(x, edge_index, batch, W1, b1, W2, b2, W3, b3, Wl, bl):
    dst3 = edge_index[1].reshape(NW, CH, K)
    edge_blk = edge_index.reshape(2, NW, NB, CB, K).transpose(1, 2, 3, 0, 4)
    zeros = jnp.zeros((NP, H), jnp.float32)
    ones16 = jnp.ones((K, H), jnp.float32)
    batch2 = batch.reshape(1, N)
    b1r, b2r, b3r = b1.reshape(1, H), b2.reshape(1, H), b3.reshape(1, H)
    blr = bl.reshape(1, C)

    degp = _sc_degree(dst3, ones16, zeros)
    t1 = _tc_matmul(x, W1)
    h1p, dis = _tc_prescale(t1, degp)

    p1 = _sc_aggregate(h1p, edge_blk, zeros)
    h2p = _tc_layer(p1, h1p, dis, b1r, W2, relu=True)

    p2 = _sc_aggregate(h2p, edge_blk, zeros)
    h3p = _tc_layer(p2, h2p, dis, b2r, W3, relu=True)

    p3 = _sc_aggregate(h3p, edge_blk, zeros)
    return _tc_pool_head(p3, h3p, dis, b3r, batch2, Wl, blr)


# speedup vs baseline: 2.9861x; 1.0119x over previous
import functools

import jax
import jax.numpy as jnp
from jax import lax
from jax.experimental import pallas as pl
from jax.experimental.pallas import tpu as pltpu
from jax.experimental.pallas import tpu_sc as plsc

N = 10000
E = 320000
F_IN = 128
H = 128
C = 10
G = 64

NC = 2
NS = 16
NW = NC * NS
EW = E // NW
K = 125
CH = EW // K
NB = 4
CB = CH // NB
NP = 10112
ROWS_W = NP // NS

_HIGH = lax.Precision.HIGHEST


def _dot(a, b):
    return lax.dot_general(a, b, (((1,), (0,)), ((), ())),
                           precision=_HIGH, preferred_element_type=jnp.float32)


def _sc_mesh():
    return plsc.VectorSubcoreMesh(core_axis_name="c", subcore_axis_name="s")


@jax.jit
def _sc_degree(dst3, ones16, zeros16):
    @functools.partial(
        pl.kernel,
        out_type=jax.ShapeDtypeStruct((NC, NP, H), jnp.float32),
        mesh=_sc_mesh(),
        scratch_types=[
            pltpu.VMEM_SHARED((NP, H), jnp.float32),
            pltpu.VMEM((CH, K), jnp.int32),
            pltpu.VMEM((K, H), jnp.float32),
        ],
    )
    def deg_kernel(dst_hbm, ones_hbm, zeros_hbm, out_hbm, acc, didx, ones_v):
        cid = lax.axis_index("c")
        sid = lax.axis_index("s")
        wid = sid * NC + cid

        pltpu.sync_copy(zeros_hbm.at[pl.ds(sid * ROWS_W, ROWS_W)],
                        acc.at[pl.ds(sid * ROWS_W, ROWS_W)])
        pltpu.sync_copy(dst_hbm.at[wid], didx)
        pltpu.sync_copy(ones_hbm, ones_v)
        plsc.subcore_barrier()

        @pl.loop(0, CH)
        def _(c):
            pltpu.sync_copy(ones_v, acc.at[didx.at[c]], add=True)

        plsc.subcore_barrier()
        pltpu.sync_copy(acc.at[pl.ds(sid * ROWS_W, ROWS_W)],
                        out_hbm.at[cid, pl.ds(sid * ROWS_W, ROWS_W)])

    return deg_kernel(dst3, ones16, zeros16)


@jax.jit
def _sc_aggregate(hp, edge_blk, zeros):
    @functools.partial(
        pl.kernel,
        out_type=jax.ShapeDtypeStruct((NC, NP, H), jnp.float32),
        mesh=_sc_mesh(),
        scratch_types=[
            pltpu.VMEM_SHARED((NP, H), jnp.float32),
            pltpu.VMEM((2, CB, K), jnp.int32),
            pltpu.VMEM((2, CB, K), jnp.int32),
            pltpu.VMEM((K, H), jnp.float32),
            pltpu.VMEM((K, H), jnp.float32),
            pltpu.SemaphoreType.DMA,
            pltpu.SemaphoreType.DMA,
            pltpu.SemaphoreType.DMA,
        ],
    )
    def agg_kernel(hp_hbm, edge_hbm, zeros_hbm, out_hbm,
                   acc, iba, ibb, bufa, bufb, sema, semb, semi):
        cid = lax.axis_index("c")
        sid = lax.axis_index("s")
        wid = sid * NC + cid

        pltpu.sync_copy(zeros_hbm.at[pl.ds(sid * ROWS_W, ROWS_W)],
                        acc.at[pl.ds(sid * ROWS_W, ROWS_W)])
        pltpu.sync_copy(edge_hbm.at[:, wid, 0], iba)
        plsc.subcore_barrier()

        def fire(ib, j, buf, sem):
            return pltpu.async_copy(hp_hbm.at[ib.at[0, j]], buf, sem)

        def wait_g(ib, j, buf, sem):
            pltpu.make_async_copy(hp_hbm.at[ib.at[0, j]], buf, sem).wait()

        def scat(ib, j, buf):
            pltpu.sync_copy(buf, acc.at[ib.at[1, j]], add=True)

        for b in range(NB):
            ib, ibn = (iba, ibb) if b % 2 == 0 else (ibb, iba)
            if b + 1 < NB:
                pltpu.async_copy(edge_hbm.at[:, wid, b + 1], ibn, semi)
            fire(ib, 0, bufa, sema)
            fire(ib, 1, bufb, semb)

            @pl.loop(0, CB, step=2)
            def _(j, ib=ib):
                wait_g(ib, j, bufa, sema)
                scat(ib, j, bufa)

                @pl.when(j + 2 < CB)
                def _():
                    fire(ib, j + 2, bufa, sema)

                @pl.when(j + 1 < CB)
                def _():
                    wait_g(ib, j + 1, bufb, semb)
                    scat(ib, j + 1, bufb)

                    @pl.when(j + 3 < CB)
                    def _():
                        fire(ib, j + 3, bufb, semb)

            if b + 1 < NB:
                pltpu.make_async_copy(edge_hbm.at[:, wid, b + 1], ibn, semi).wait()

        plsc.subcore_barrier()
        pltpu.sync_copy(acc.at[pl.ds(sid * ROWS_W, ROWS_W)],
                        out_hbm.at[cid, pl.ds(sid * ROWS_W, ROWS_W)])

    return agg_kernel(hp, edge_blk, zeros)


_R = 1000


def _tc_matmul(x, W):
    def body(x_ref, w_ref, o_ref):
        o_ref[...] = _dot(x_ref[...], w_ref[...])

    return pl.pallas_call(
        body,
        grid=(N // _R,),
        in_specs=[pl.BlockSpec((_R, F_IN), lambda i: (i, 0)),
                  pl.BlockSpec((F_IN, H), lambda i: (0, 0))],
        out_specs=pl.BlockSpec((_R, H), lambda i: (i, 0)),
        out_shape=jax.ShapeDtypeStruct((N, H), jnp.float32),
    )(x, W)


def _tc_prescale(t1, degp):
    def body(t_ref, d_ref, hp_ref, dis_ref):
        deg = d_ref[0, :, 0:1] + d_ref[1, :, 0:1] + 1.0
        dis = lax.rsqrt(deg)
        dis_ref[...] = dis
        hp_ref[...] = t_ref[...] * dis

    return pl.pallas_call(
        body,
        grid=(N // _R,),
        in_specs=[pl.BlockSpec((_R, H), lambda i: (i, 0)),
                  pl.BlockSpec((NC, _R, H), lambda i: (0, i, 0))],
        out_specs=[pl.BlockSpec((_R, H), lambda i: (i, 0)),
                   pl.BlockSpec((_R, 1), lambda i: (i, 0))],
        out_shape=[jax.ShapeDtypeStruct((N, H), jnp.float32),
                   jax.ShapeDtypeStruct((N, 1), jnp.float32)],
    )(t1, degp)


def _tc_layer(parts, hp, dis, b, W, relu=True):
    def body(p_ref, hp_ref, dis_ref, b_ref, w_ref, o_ref):
        dis = dis_ref[...]
        z = dis * (p_ref[0] + p_ref[1] + hp_ref[...]) + b_ref[...]
        if relu:
            z = jnp.maximum(z, 0.0)
        o_ref[...] = _dot(z, w_ref[...]) * dis

    return pl.pallas_call(
        body,
        grid=(N // _R,),
        in_specs=[pl.BlockSpec((NC, _R, H), lambda i: (0, i, 0)),
                  pl.BlockSpec((_R, H), lambda i: (i, 0)),
                  pl.BlockSpec((_R, 1), lambda i: (i, 0)),
                  pl.BlockSpec((1, H), lambda i: (0, 0)),
                  pl.BlockSpec((H, H), lambda i: (0, 0))],
        out_specs=pl.BlockSpec((_R, H), lambda i: (i, 0)),
        out_shape=jax.ShapeDtypeStruct((N, H), jnp.float32),
    )(parts, hp, dis, b, W)


def _tc_pool_head(parts, hp, dis, b3, batch2, Wl, bl):
    def body(p_ref, hp_ref, dis_ref, b_ref, bat_ref, wl_ref, bl_ref, o_ref):
        z = dis_ref[...] * (p_ref[0] + p_ref[1] + hp_ref[...]) + b_ref[...]
        gids = lax.broadcasted_iota(jnp.int32, (G, N), 0)
        mask = (bat_ref[...] == gids).astype(jnp.float32)
        sums = _dot(mask, z)
        counts = jnp.sum(mask, axis=1, keepdims=True)
        pooled = sums / jnp.maximum(counts, 1.0)
        o_ref[...] = _dot(pooled, wl_ref[...]) + bl_ref[...]

    return pl.pallas_call(
        body,
        grid=(1,),
        in_specs=[pl.BlockSpec((NC, N, H), lambda i: (0, 0, 0)),
                  pl.BlockSpec((N, H), lambda i: (0, 0)),
                  pl.BlockSpec((N, 1), lambda i: (0, 0)),
                  pl.BlockSpec((1, H), lambda i: (0, 0)),
                  pl.BlockSpec((1, N), lambda i: (0, 0)),
                  pl.BlockSpec((H, C), lambda i: (0, 0)),
                  pl.BlockSpec((1, C), lambda i: (0, 0))],
        out_specs=pl.BlockSpec((G, C), lambda i: (0, 0)),
        out_shape=jax.ShapeDtypeStruct((G, C), jnp.float32),
    )(parts, hp, dis, b3, batch2, Wl, bl)


def kernel(x, edge_index, batch, W1, b1, W2, b2, W3, b3, Wl, bl):
    dst3 = edge_index[1].reshape(NW, CH, K)
    edge_blk = edge_index.reshape(2, NW, NB, CB, K)
    zeros = jnp.zeros((NP, H), jnp.float32)
    ones16 = jnp.ones((K, H), jnp.float32)
    batch2 = batch.reshape(1, N)
    b1r, b2r, b3r = b1.reshape(1, H), b2.reshape(1, H), b3.reshape(1, H)
    blr = bl.reshape(1, C)

    degp = _sc_degree(dst3, ones16, zeros)
    t1 = _tc_matmul(x, W1)
    h1p, dis = _tc_prescale(t1, degp)

    p1 = _sc_aggregate(h1p, edge_blk, zeros)
    h2p = _tc_layer(p1, h1p, dis, b1r, W2, relu=True)

    p2 = _sc_aggregate(h2p, edge_blk, zeros)
    h3p = _tc_layer(p2, h2p, dis, b2r, W3, relu=True)

    p3 = _sc_aggregate(h3p, edge_blk, zeros)
    return _tc_pool_head(p3, h3p, dis, b3r, batch2, Wl, blr)


# speedup vs baseline: 3.0008x; 1.0049x over previous
import functools

import jax
import jax.numpy as jnp
from jax import lax
from jax.experimental import pallas as pl
from jax.experimental.pallas import tpu as pltpu
from jax.experimental.pallas import tpu_sc as plsc

N = 10000
E = 320000
F_IN = 128
H = 128
C = 10
G = 64

NC = 2
NS = 16
NW = NC * NS
EW = E // NW
K = 125
CH = EW // K
NB = 4
CB = CH // NB
NP = 10112
ROWS_W = NP // NS

_HIGH = lax.Precision.HIGHEST


def _dot(a, b, precision=_HIGH):
    return lax.dot_general(a, b, (((1,), (0,)), ((), ())),
                           precision=precision,
                           preferred_element_type=jnp.float32)


def _sc_mesh():
    return plsc.VectorSubcoreMesh(core_axis_name="c", subcore_axis_name="s")


@jax.jit
def _sc_degree(dst3, ones16, zeros16):
    @functools.partial(
        pl.kernel,
        out_type=jax.ShapeDtypeStruct((NC, NP, H), jnp.float32),
        mesh=_sc_mesh(),
        scratch_types=[
            pltpu.VMEM_SHARED((NP, H), jnp.float32),
            pltpu.VMEM((CH, K), jnp.int32),
            pltpu.VMEM((K, H), jnp.float32),
        ],
    )
    def deg_kernel(dst_hbm, ones_hbm, zeros_hbm, out_hbm, acc, didx, ones_v):
        cid = lax.axis_index("c")
        sid = lax.axis_index("s")
        wid = sid * NC + cid

        pltpu.sync_copy(zeros_hbm.at[pl.ds(sid * ROWS_W, ROWS_W)],
                        acc.at[pl.ds(sid * ROWS_W, ROWS_W)])
        pltpu.sync_copy(dst_hbm.at[wid], didx)
        pltpu.sync_copy(ones_hbm, ones_v)
        plsc.subcore_barrier()

        @pl.loop(0, CH)
        def _(c):
            pltpu.sync_copy(ones_v, acc.at[didx.at[c]], add=True)

        plsc.subcore_barrier()
        pltpu.sync_copy(acc.at[pl.ds(sid * ROWS_W, ROWS_W)],
                        out_hbm.at[cid, pl.ds(sid * ROWS_W, ROWS_W)])

    return deg_kernel(dst3, ones16, zeros16)


@jax.jit
def _sc_aggregate(hp, edge_blk, zeros):
    @functools.partial(
        pl.kernel,
        out_type=jax.ShapeDtypeStruct((NC, NP, H), jnp.float32),
        mesh=_sc_mesh(),
        scratch_types=[
            pltpu.VMEM_SHARED((NP, H), jnp.float32),
            pltpu.VMEM((2, CB, K), jnp.int32),
            pltpu.VMEM((2, CB, K), jnp.int32),
            pltpu.VMEM((K, H), jnp.float32),
            pltpu.VMEM((K, H), jnp.float32),
            pltpu.SemaphoreType.DMA,
            pltpu.SemaphoreType.DMA,
            pltpu.SemaphoreType.DMA,
        ],
    )
    def agg_kernel(hp_hbm, edge_hbm, zeros_hbm, out_hbm,
                   acc, iba, ibb, bufa, bufb, sema, semb, semi):
        cid = lax.axis_index("c")
        sid = lax.axis_index("s")
        wid = sid * NC + cid

        pltpu.sync_copy(zeros_hbm.at[pl.ds(sid * ROWS_W, ROWS_W)],
                        acc.at[pl.ds(sid * ROWS_W, ROWS_W)])
        pltpu.sync_copy(edge_hbm.at[:, wid, 0], iba)
        plsc.subcore_barrier()

        def fire(ib, j, buf, sem):
            return pltpu.async_copy(hp_hbm.at[ib.at[0, j]], buf, sem)

        def wait_g(ib, j, buf, sem):
            pltpu.make_async_copy(hp_hbm.at[ib.at[0, j]], buf, sem).wait()

        def scat(ib, j, buf):
            pltpu.sync_copy(buf, acc.at[ib.at[1, j]], add=True)

        for b in range(NB):
            ib, ibn = (iba, ibb) if b % 2 == 0 else (ibb, iba)
            if b + 1 < NB:
                pltpu.async_copy(edge_hbm.at[:, wid, b + 1], ibn, semi)
            fire(ib, 0, bufa, sema)
            fire(ib, 1, bufb, semb)

            @pl.loop(0, CB, step=2)
            def _(j, ib=ib):
                wait_g(ib, j, bufa, sema)
                scat(ib, j, bufa)

                @pl.when(j + 2 < CB)
                def _():
                    fire(ib, j + 2, bufa, sema)

                @pl.when(j + 1 < CB)
                def _():
                    wait_g(ib, j + 1, bufb, semb)
                    scat(ib, j + 1, bufb)

                    @pl.when(j + 3 < CB)
                    def _():
                        fire(ib, j + 3, bufb, semb)

            if b + 1 < NB:
                pltpu.make_async_copy(edge_hbm.at[:, wid, b + 1], ibn, semi).wait()

        plsc.subcore_barrier()
        pltpu.sync_copy(acc.at[pl.ds(sid * ROWS_W, ROWS_W)],
                        out_hbm.at[cid, pl.ds(sid * ROWS_W, ROWS_W)])

    return agg_kernel(hp, edge_blk, zeros)


_R = 1000


def _tc_matmul(x, W):
    def body(x_ref, w_ref, o_ref):
        o_ref[...] = _dot(x_ref[...], w_ref[...])

    return pl.pallas_call(
        body,
        grid=(N // _R,),
        in_specs=[pl.BlockSpec((_R, F_IN), lambda i: (i, 0)),
                  pl.BlockSpec((F_IN, H), lambda i: (0, 0))],
        out_specs=pl.BlockSpec((_R, H), lambda i: (i, 0)),
        out_shape=jax.ShapeDtypeStruct((N, H), jnp.float32),
    )(x, W)


def _tc_prescale(t1, degp):
    def body(t_ref, d_ref, hp_ref, dis_ref):
        deg = d_ref[0, :, 0:1] + d_ref[1, :, 0:1] + 1.0
        dis = lax.rsqrt(deg)
        dis_ref[...] = dis
        hp_ref[...] = t_ref[...] * dis

    return pl.pallas_call(
        body,
        grid=(N // _R,),
        in_specs=[pl.BlockSpec((_R, H), lambda i: (i, 0)),
                  pl.BlockSpec((NC, _R, H), lambda i: (0, i, 0))],
        out_specs=[pl.BlockSpec((_R, H), lambda i: (i, 0)),
                   pl.BlockSpec((_R, 1), lambda i: (i, 0))],
        out_shape=[jax.ShapeDtypeStruct((N, H), jnp.float32),
                   jax.ShapeDtypeStruct((N, 1), jnp.float32)],
    )(t1, degp)


def _tc_layer(parts, hp, dis, b, W, relu=True):
    def body(p_ref, hp_ref, dis_ref, b_ref, w_ref, o_ref):
        dis = dis_ref[...]
        z = dis * (p_ref[0] + p_ref[1] + hp_ref[...]) + b_ref[...]
        if relu:
            z = jnp.maximum(z, 0.0)
        o_ref[...] = _dot(z, w_ref[...]) * dis

    return pl.pallas_call(
        body,
        grid=(N // _R,),
        in_specs=[pl.BlockSpec((NC, _R, H), lambda i: (0, i, 0)),
                  pl.BlockSpec((_R, H), lambda i: (i, 0)),
                  pl.BlockSpec((_R, 1), lambda i: (i, 0)),
                  pl.BlockSpec((1, H), lambda i: (0, 0)),
                  pl.BlockSpec((H, H), lambda i: (0, 0))],
        out_specs=pl.BlockSpec((_R, H), lambda i: (i, 0)),
        out_shape=jax.ShapeDtypeStruct((N, H), jnp.float32),
    )(parts, hp, dis, b, W)


def _tc_pool_head(parts, hp, dis, b3, batch2, Wl, bl):
    def body(p_ref, hp_ref, dis_ref, b_ref, bat_ref, wl_ref, bl_ref, o_ref):
        z = dis_ref[...] * (p_ref[0] + p_ref[1] + hp_ref[...]) + b_ref[...]
        gids = lax.broadcasted_iota(jnp.int32, (G, N), 0)
        mask = (bat_ref[...] == gids).astype(jnp.float32)
        sums = _dot(mask, z, precision=lax.Precision.DEFAULT)
        counts = jnp.sum(mask, axis=1, keepdims=True)
        pooled = sums / jnp.maximum(counts, 1.0)
        o_ref[...] = _dot(pooled, wl_ref[...]) + bl_ref[...]

    return pl.pallas_call(
        body,
        grid=(1,),
        in_specs=[pl.BlockSpec((NC, N, H), lambda i: (0, 0, 0)),
                  pl.BlockSpec((N, H), lambda i: (0, 0)),
                  pl.BlockSpec((N, 1), lambda i: (0, 0)),
                  pl.BlockSpec((1, H), lambda i: (0, 0)),
                  pl.BlockSpec((1, N), lambda i: (0, 0)),
                  pl.BlockSpec((H, C), lambda i: (0, 0)),
                  pl.BlockSpec((1, C), lambda i: (0, 0))],
        out_specs=pl.BlockSpec((G, C), lambda i: (0, 0)),
        out_shape=jax.ShapeDtypeStruct((G, C), jnp.float32),
    )(parts, hp, dis, b3, batch2, Wl, bl)


def kernel(x, edge_index, batch, W1, b1, W2, b2, W3, b3, Wl, bl):
    dst3 = edge_index[1].reshape(NW, CH, K)
    edge_blk = edge_index.reshape(2, NW, NB, CB, K)
    zeros = jnp.zeros((NP, H), jnp.float32)
    ones16 = jnp.ones((K, H), jnp.float32)
    batch2 = batch.reshape(1, N)
    b1r, b2r, b3r = b1.reshape(1, H), b2.reshape(1, H), b3.reshape(1, H)
    blr = bl.reshape(1, C)

    degp = _sc_degree(dst3, ones16, zeros)
    t1 = _tc_matmul(x, W1)
    h1p, dis = _tc_prescale(t1, degp)

    p1 = _sc_aggregate(h1p, edge_blk, zeros)
    h2p = _tc_layer(p1, h1p, dis, b1r, W2, relu=True)

    p2 = _sc_aggregate(h2p, edge_blk, zeros)
    h3p = _tc_layer(p2, h2p, dis, b2r, W3, relu=True)

    p3 = _sc_aggregate(h3p, edge_blk, zeros)
    return _tc_pool_head(p3, h3p, dis, b3r, batch2, Wl, blr)
